# baseline (speedup 1.0000x reference)
import functools

import jax
import jax.numpy as jnp
from jax import lax
from jax.experimental import pallas as pl
from jax.experimental.pallas import tpu as pltpu
from jax.experimental.pallas import tpu_sc as plsc

N = 10000
K = 16
D_IN = 16
HEADS = 2
DIM_HEAD = 8
HIDDEN = HEADS * DIM_HEAD
KV_DIM = HIDDEN * 2
MID = 128
EDGES = N * K
EPS = 1e-5

NUM_CORES = 2
NUM_SUBCORES = 16
NW = NUM_CORES * NUM_SUBCORES
PER_W = EDGES // NW
CW = 125
CH = PER_W // CW

T = 200
GRID = N // T
ET = T * K


def _sc_gather(table, idx3):
    mesh = plsc.VectorSubcoreMesh(core_axis_name="c", subcore_axis_name="s")

    @functools.partial(
        pl.kernel,
        mesh=mesh,
        out_type=jax.ShapeDtypeStruct((EDGES, D_IN), jnp.float32),
        scratch_types=[
            pltpu.VMEM((CH, CW), jnp.int32),
            pltpu.VMEM((PER_W, D_IN), jnp.float32),
            pltpu.SemaphoreType.DMA,
        ],
        compiler_params=pltpu.CompilerParams(use_tc_tiling_on_sc=False),
    )
    def gather_kernel(table_hbm, idx_hbm, out_hbm, idx_v, rows_v, sem):
        wid = lax.axis_index("s") * NUM_CORES + lax.axis_index("c")
        pltpu.sync_copy(idx_hbm.at[wid], idx_v)

        def body(ci, carry):
            pltpu.async_copy(
                table_hbm.at[idx_v.at[ci]],
                rows_v.at[pl.ds(ci * CW, CW)],
                sem,
            ).wait()
            return carry

        lax.fori_loop(0, CH, body, 0)
        pltpu.sync_copy(rows_v, out_hbm.at[pl.ds(wid * PER_W, PER_W)])

    return gather_kernel(table, idx3)


def _tc_body(rd_ref, mf_ref, bs_ref, feat_ref, xg_ref, vecs_ref, w2_ref,
             w3_ref, eb_ref, s_ref, wq_ref, wout_ref, out_ref):
    f32 = jnp.float32
    rd = rd_ref[...]
    am = vecs_ref[0:1, :].reshape(1, 1, MID)
    cm = vecs_ref[1:2, :].reshape(1, 1, MID)
    g1 = vecs_ref[2:3, :].reshape(1, 1, MID)
    be1 = vecs_ref[3:4, :].reshape(1, 1, MID)
    pre = rd[:, :, None] * am + cm
    var1 = jnp.mean(pre * pre, axis=-1, keepdims=True)
    h1 = jnp.maximum(pre * lax.rsqrt(var1 + EPS) * g1 + be1, 0.0)
    h1f = h1.reshape(ET, MID)
    h2p = jnp.dot(h1f, w2_ref[...], preferred_element_type=f32)
    h2p = h2p + vecs_ref[4:5, :]
    m2 = jnp.mean(h2p, axis=-1, keepdims=True)
    d2 = h2p - m2
    var2 = jnp.mean(d2 * d2, axis=-1, keepdims=True)
    h2 = jnp.maximum(d2 * lax.rsqrt(var2 + EPS) * vecs_ref[5:6, :]
                     + vecs_ref[6:7, :], 0.0)
    y = jnp.dot(h2, w3_ref[...], preferred_element_type=f32)
    xg = xg_ref[...]
    mb = jnp.dot(xg, eb_ref[...], preferred_element_type=f32,
                 precision=lax.Precision.HIGHEST)
    z = y * mb[:, :KV_DIM * D_IN]
    kv = jnp.dot(z, s_ref[...], preferred_element_type=f32,
                 precision=lax.Precision.HIGHEST)
    kv = kv + mb[:, KV_DIM * D_IN:]
    kv3 = kv.reshape(T, K, KV_DIM) * bs_ref[...][:, :, None]
    q = jnp.dot(feat_ref[...], wq_ref[...], preferred_element_type=f32)
    mf = mf_ref[...]
    scale = DIM_HEAD ** -0.5
    neg = -jnp.finfo(f32).max
    outs = []
    for h in range(HEADS):
        qh = q[:, h * DIM_HEAD:(h + 1) * DIM_HEAD]
        kh = kv3[:, :, h * 2 * DIM_HEAD:h * 2 * DIM_HEAD + DIM_HEAD]
        vh = kv3[:, :, h * 2 * DIM_HEAD + DIM_HEAD:(h + 1) * 2 * DIM_HEAD]
        sim = jnp.sum(qh[:, None, :] * kh, axis=-1) * scale
        sim = jnp.where(mf != 0.0, sim, neg)
        mx = jnp.max(sim, axis=-1, keepdims=True)
        ex = jnp.exp(sim - mx)
        attn = ex / jnp.sum(ex, axis=-1, keepdims=True)
        outs.append(jnp.sum(attn[:, :, None] * vh, axis=1))
    out16 = jnp.concatenate(outs, axis=-1)
    out_ref[...] = jnp.dot(out16, wout_ref[...], preferred_element_type=f32)


def _tc_fused(rd, mf, bs, feat, xg, vecs, w2, w3p, ebmat, smat, wq, wout):
    node_spec = pl.BlockSpec((T, K), lambda i: (i, 0))
    full = lambda shape: pl.BlockSpec(shape, lambda i: tuple(0 for _ in shape))
    return pl.pallas_call(
        _tc_body,
        grid=(GRID,),
        in_specs=[
            node_spec,
            node_spec,
            node_spec,
            pl.BlockSpec((T, D_IN), lambda i: (i, 0)),
            pl.BlockSpec((ET, D_IN), lambda i: (i, 0)),
            full((8, MID)),
            full((MID, MID)),
            full((MID, KV_DIM * D_IN)),
            full((D_IN, KV_DIM * D_IN + KV_DIM)),
            full((KV_DIM * D_IN, KV_DIM)),
            full((D_IN, HIDDEN)),
            full((HIDDEN, D_IN)),
        ],
        out_specs=pl.BlockSpec((T, D_IN), lambda i: (i, 0)),
        out_shape=jax.ShapeDtypeStruct((N, D_IN), jnp.float32),
    )(rd, mf, bs, feat, xg, vecs, w2, w3p, ebmat, smat, wq, wout)


def kernel(features_0, neighbor_indices, neighbor_mask, rel_dist, basis_0_0,
           Wq, W1, b1, g1, be1, W2, b2, g2, be2, W3, b3, Wout):
    feat = features_0.reshape(N, D_IN)
    idx3 = neighbor_indices.reshape(EDGES).astype(jnp.int32).reshape(NW, CH, CW)
    rd = rel_dist.reshape(N, K)
    mf = neighbor_mask.reshape(N, K).astype(jnp.float32)
    bs = basis_0_0.reshape(N, K)

    a = W1.reshape(MID)
    am = a - jnp.mean(a)
    cm = b1 - jnp.mean(b1)
    vecs = jnp.concatenate(
        [jnp.stack([am, cm, g1, be1, b2, g2, be2], axis=0),
         jnp.zeros((1, MID), jnp.float32)], axis=0)
    w3p = W3.reshape(MID, KV_DIM, D_IN).transpose(0, 2, 1).reshape(
        MID, KV_DIM * D_IN)
    b3m = b3.reshape(KV_DIM, D_IN).T
    emat = jnp.kron(jnp.eye(D_IN, dtype=jnp.float32),
                    jnp.ones((1, KV_DIM), jnp.float32))
    ebmat = jnp.concatenate([emat, b3m], axis=1)
    smat = jnp.kron(jnp.ones((D_IN, 1), jnp.float32),
                    jnp.eye(KV_DIM, dtype=jnp.float32))

    gathered = _sc_gather(feat, idx3)
    out = _tc_fused(rd, mf, bs, feat, gathered, vecs, W2, w3p, ebmat, smat,
                    Wq, Wout)
    return out.reshape(1, N, HIDDEN, 1)


# speedup vs baseline: 1.6527x; 1.6527x over previous
import functools

import jax
import jax.numpy as jnp
from jax import lax
from jax.experimental import pallas as pl
from jax.experimental.pallas import tpu as pltpu
from jax.experimental.pallas import tpu_sc as plsc

N = 10000
K = 16
D_IN = 16
HEADS = 2
DIM_HEAD = 8
HIDDEN = HEADS * DIM_HEAD
KV_DIM = HIDDEN * 2
MID = 128
EDGES = N * K
EPS = 1e-5

NUM_CORES = 2
NUM_SUBCORES = 16
NW = NUM_CORES * NUM_SUBCORES
PER_W = EDGES // NW
CW = 125
CH = PER_W // CW

T = 200
GRID = N // T
ET = T * K


def _sc_gather(table, idx3):
    mesh = plsc.VectorSubcoreMesh(core_axis_name="c", subcore_axis_name="s")

    @functools.partial(
        pl.kernel,
        mesh=mesh,
        out_type=jax.ShapeDtypeStruct((EDGES, D_IN), jnp.float32),
        scratch_types=[
            pltpu.VMEM((CH, CW), jnp.int32),
            pltpu.VMEM((PER_W, D_IN), jnp.float32),
            pltpu.SemaphoreType.DMA,
        ],
        compiler_params=pltpu.CompilerParams(use_tc_tiling_on_sc=False),
    )
    def gather_kernel(table_hbm, idx_hbm, out_hbm, idx_v, rows_v, sem):
        wid = lax.axis_index("s") * NUM_CORES + lax.axis_index("c")
        pltpu.sync_copy(idx_hbm.at[wid], idx_v)

        def body(ci, carry):
            pltpu.async_copy(
                table_hbm.at[idx_v.at[ci]],
                rows_v.at[pl.ds(ci * CW, CW)],
                sem,
            ).wait()
            return carry

        lax.fori_loop(0, CH, body, 0)
        pltpu.sync_copy(rows_v, out_hbm.at[pl.ds(wid * PER_W, PER_W)])

    return gather_kernel(table, idx3)


def _tc_body(rd_ref, mf_ref, bs_ref, feat_ref, xg_ref, vecs_ref, w2_ref,
             w3_ref, eb_ref, s_ref, wq_ref, wout_ref, out_ref):
    f32 = jnp.float32
    rd = rd_ref[...]
    am = vecs_ref[0:1, :].reshape(1, 1, MID)
    cm = vecs_ref[1:2, :].reshape(1, 1, MID)
    g1 = vecs_ref[2:3, :].reshape(1, 1, MID)
    be1 = vecs_ref[3:4, :].reshape(1, 1, MID)
    pre = rd[:, :, None] * am + cm
    var1 = jnp.mean(pre * pre, axis=-1, keepdims=True)
    h1 = jnp.maximum(pre * lax.rsqrt(var1 + EPS) * g1 + be1, 0.0)
    h1f = h1.reshape(ET, MID)
    h2p = jnp.dot(h1f, w2_ref[...], preferred_element_type=f32)
    h2p = h2p + vecs_ref[4:5, :]
    m2 = jnp.mean(h2p, axis=-1, keepdims=True)
    d2 = h2p - m2
    var2 = jnp.mean(d2 * d2, axis=-1, keepdims=True)
    h2 = jnp.maximum(d2 * lax.rsqrt(var2 + EPS) * vecs_ref[5:6, :]
                     + vecs_ref[6:7, :], 0.0)
    y = jnp.dot(h2, w3_ref[...], preferred_element_type=f32)
    xg = xg_ref[...]
    eb = eb_ref[...]
    xh = lax.convert_element_type(
        lax.convert_element_type(xg, jnp.bfloat16), f32)
    xl = xg - xh
    mb = (jnp.dot(xh, eb, preferred_element_type=f32)
          + jnp.dot(xl, eb, preferred_element_type=f32))
    z = y * mb[:, :KV_DIM * D_IN]
    zh = lax.convert_element_type(
        lax.convert_element_type(z, jnp.bfloat16), f32)
    zl = z - zh
    sm = s_ref[...]
    kv = (jnp.dot(zh, sm, preferred_element_type=f32)
          + jnp.dot(zl, sm, preferred_element_type=f32))
    kv = kv + mb[:, KV_DIM * D_IN:]
    kv3 = kv.reshape(T, K, KV_DIM) * bs_ref[...][:, :, None]
    q = jnp.dot(feat_ref[...], wq_ref[...], preferred_element_type=f32)
    mf = mf_ref[...]
    scale = DIM_HEAD ** -0.5
    neg = -jnp.finfo(f32).max
    outs = []
    for h in range(HEADS):
        qh = q[:, h * DIM_HEAD:(h + 1) * DIM_HEAD]
        kh = kv3[:, :, h * 2 * DIM_HEAD:h * 2 * DIM_HEAD + DIM_HEAD]
        vh = kv3[:, :, h * 2 * DIM_HEAD + DIM_HEAD:(h + 1) * 2 * DIM_HEAD]
        sim = jnp.sum(qh[:, None, :] * kh, axis=-1) * scale
        sim = jnp.where(mf != 0.0, sim, neg)
        mx = jnp.max(sim, axis=-1, keepdims=True)
        ex = jnp.exp(sim - mx)
        attn = ex / jnp.sum(ex, axis=-1, keepdims=True)
        outs.append(jnp.sum(attn[:, :, None] * vh, axis=1))
    out16 = jnp.concatenate(outs, axis=-1)
    out_ref[...] = jnp.dot(out16, wout_ref[...], preferred_element_type=f32)


def _tc_fused(rd, mf, bs, feat, xg, vecs, w2, w3p, ebmat, smat, wq, wout):
    node_spec = pl.BlockSpec((T, K), lambda i: (i, 0))
    full = lambda shape: pl.BlockSpec(shape, lambda i: tuple(0 for _ in shape))
    return pl.pallas_call(
        _tc_body,
        grid=(GRID,),
        in_specs=[
            node_spec,
            node_spec,
            node_spec,
            pl.BlockSpec((T, D_IN), lambda i: (i, 0)),
            pl.BlockSpec((ET, D_IN), lambda i: (i, 0)),
            full((8, MID)),
            full((MID, MID)),
            full((MID, KV_DIM * D_IN)),
            full((D_IN, KV_DIM * D_IN + KV_DIM)),
            full((KV_DIM * D_IN, KV_DIM)),
            full((D_IN, HIDDEN)),
            full((HIDDEN, D_IN)),
        ],
        out_specs=pl.BlockSpec((T, D_IN), lambda i: (i, 0)),
        out_shape=jax.ShapeDtypeStruct((N, D_IN), jnp.float32),
    )(rd, mf, bs, feat, xg, vecs, w2, w3p, ebmat, smat, wq, wout)


def kernel(features_0, neighbor_indices, neighbor_mask, rel_dist, basis_0_0,
           Wq, W1, b1, g1, be1, W2, b2, g2, be2, W3, b3, Wout):
    feat = features_0.reshape(N, D_IN)
    idx3 = neighbor_indices.reshape(EDGES).astype(jnp.int32).reshape(NW, CH, CW)
    rd = rel_dist.reshape(N, K)
    mf = neighbor_mask.reshape(N, K).astype(jnp.float32)
    bs = basis_0_0.reshape(N, K)

    a = W1.reshape(MID)
    am = a - jnp.mean(a)
    cm = b1 - jnp.mean(b1)
    vecs = jnp.concatenate(
        [jnp.stack([am, cm, g1, be1, b2, g2, be2], axis=0),
         jnp.zeros((1, MID), jnp.float32)], axis=0)
    w3p = W3.reshape(MID, KV_DIM, D_IN).transpose(0, 2, 1).reshape(
        MID, KV_DIM * D_IN)
    b3m = b3.reshape(KV_DIM, D_IN).T
    emat = jnp.kron(jnp.eye(D_IN, dtype=jnp.float32),
                    jnp.ones((1, KV_DIM), jnp.float32))
    ebmat = jnp.concatenate([emat, b3m], axis=1)
    smat = jnp.kron(jnp.ones((D_IN, 1), jnp.float32),
                    jnp.eye(KV_DIM, dtype=jnp.float32))

    gathered = _sc_gather(feat, idx3)
    out = _tc_fused(rd, mf, bs, feat, gathered, vecs, W2, w3p, ebmat, smat,
                    Wq, Wout)
    return out.reshape(1, N, HIDDEN, 1)


# speedup vs baseline: 2.4862x; 1.5043x over previous
import functools

import jax
import jax.numpy as jnp
from jax import lax
from jax.experimental import pallas as pl
from jax.experimental.pallas import tpu as pltpu
from jax.experimental.pallas import tpu_sc as plsc

N = 10000
K = 16
D_IN = 16
HEADS = 2
DIM_HEAD = 8
HIDDEN = HEADS * DIM_HEAD
KV_DIM = HIDDEN * 2
MID = 128
EDGES = N * K
EPS = 1e-5

NUM_CORES = 2
NUM_SUBCORES = 16
NW = NUM_CORES * NUM_SUBCORES
PER_W = EDGES // NW
CW = 125
CH = PER_W // CW

T = 200
GRID = N // T
ET = T * K


def _sc_gather(table, idx3):
    mesh = plsc.VectorSubcoreMesh(core_axis_name="c", subcore_axis_name="s")

    @functools.partial(
        pl.kernel,
        mesh=mesh,
        out_type=jax.ShapeDtypeStruct((EDGES, D_IN), jnp.float32),
        scratch_types=[
            pltpu.VMEM((CH, CW), jnp.int32),
            pltpu.VMEM((PER_W, D_IN), jnp.float32),
            pltpu.SemaphoreType.DMA,
        ],
        compiler_params=pltpu.CompilerParams(use_tc_tiling_on_sc=False),
    )
    def gather_kernel(table_hbm, idx_hbm, out_hbm, idx_v, rows_v, sem):
        wid = lax.axis_index("s") * NUM_CORES + lax.axis_index("c")
        pltpu.sync_copy(idx_hbm.at[wid], idx_v)

        def body(ci, carry):
            pltpu.async_copy(
                table_hbm.at[idx_v.at[ci]],
                rows_v.at[pl.ds(ci * CW, CW)],
                sem,
            ).wait()
            return carry

        lax.fori_loop(0, CH, body, 0)
        pltpu.sync_copy(rows_v, out_hbm.at[pl.ds(wid * PER_W, PER_W)])

    return gather_kernel(table, idx3)


def _tc_body(rd_ref, mf_ref, bs_ref, feat_ref, xg_ref, vecs_ref, w2_ref,
             w3_ref, eb_ref, wq_ref, wout_ref, out_ref):
    f32 = jnp.float32
    rd = rd_ref[...]
    am = vecs_ref[0:1, :].reshape(1, 1, MID)
    cm = vecs_ref[1:2, :].reshape(1, 1, MID)
    g1 = vecs_ref[2:3, :].reshape(1, 1, MID)
    be1 = vecs_ref[3:4, :].reshape(1, 1, MID)
    pre = rd[:, :, None] * am + cm
    var1 = jnp.mean(pre * pre, axis=-1, keepdims=True)
    h1 = jnp.maximum(pre * lax.rsqrt(var1 + EPS) * g1 + be1, 0.0)
    h1f = h1.reshape(ET, MID)
    h2p = jnp.dot(h1f, w2_ref[...], preferred_element_type=f32)
    h2p = h2p + vecs_ref[4:5, :]
    m2 = jnp.mean(h2p, axis=-1, keepdims=True)
    d2 = h2p - m2
    var2 = jnp.mean(d2 * d2, axis=-1, keepdims=True)
    h2 = jnp.maximum(d2 * lax.rsqrt(var2 + EPS) * vecs_ref[5:6, :]
                     + vecs_ref[6:7, :], 0.0)
    y = jnp.dot(h2, w3_ref[...], preferred_element_type=f32)
    xg = xg_ref[...]
    eb = eb_ref[...]
    xh = lax.convert_element_type(
        lax.convert_element_type(xg, jnp.bfloat16), f32)
    xl = xg - xh
    mb = (jnp.dot(xh, eb, preferred_element_type=f32)
          + jnp.dot(xl, eb, preferred_element_type=f32))
    z = y * mb[:, :KV_DIM * D_IN]
    k128 = ((z[:, 0:MID] + z[:, MID:2 * MID])
            + (z[:, 2 * MID:3 * MID] + z[:, 3 * MID:4 * MID]))
    kv = ((k128[:, 0:KV_DIM] + k128[:, KV_DIM:2 * KV_DIM])
          + (k128[:, 2 * KV_DIM:3 * KV_DIM] + k128[:, 3 * KV_DIM:4 * KV_DIM]))
    kv = kv + mb[:, KV_DIM * D_IN:]
    kv3 = kv.reshape(T, K, KV_DIM) * bs_ref[...][:, :, None]
    q = jnp.dot(feat_ref[...], wq_ref[...], preferred_element_type=f32)
    mf = mf_ref[...]
    scale = DIM_HEAD ** -0.5
    neg = -jnp.finfo(f32).max
    outs = []
    for h in range(HEADS):
        qh = q[:, h * DIM_HEAD:(h + 1) * DIM_HEAD]
        kh = kv3[:, :, h * 2 * DIM_HEAD:h * 2 * DIM_HEAD + DIM_HEAD]
        vh = kv3[:, :, h * 2 * DIM_HEAD + DIM_HEAD:(h + 1) * 2 * DIM_HEAD]
        sim = jnp.sum(qh[:, None, :] * kh, axis=-1) * scale
        sim = jnp.where(mf != 0.0, sim, neg)
        mx = jnp.max(sim, axis=-1, keepdims=True)
        ex = jnp.exp(sim - mx)
        attn = ex / jnp.sum(ex, axis=-1, keepdims=True)
        outs.append(jnp.sum(attn[:, :, None] * vh, axis=1))
    out16 = jnp.concatenate(outs, axis=-1)
    out_ref[...] = jnp.dot(out16, wout_ref[...], preferred_element_type=f32)


def _tc_fused(rd, mf, bs, feat, xg, vecs, w2, w3p, ebmat, wq, wout):
    node_spec = pl.BlockSpec((T, K), lambda i: (i, 0))
    full = lambda shape: pl.BlockSpec(shape, lambda i: tuple(0 for _ in shape))
    return pl.pallas_call(
        _tc_body,
        grid=(GRID,),
        in_specs=[
            node_spec,
            node_spec,
            node_spec,
            pl.BlockSpec((T, D_IN), lambda i: (i, 0)),
            pl.BlockSpec((ET, D_IN), lambda i: (i, 0)),
            full((8, MID)),
            full((MID, MID)),
            full((MID, KV_DIM * D_IN)),
            full((D_IN, KV_DIM * D_IN + KV_DIM)),
            full((D_IN, HIDDEN)),
            full((HIDDEN, D_IN)),
        ],
        out_specs=pl.BlockSpec((T, D_IN), lambda i: (i, 0)),
        out_shape=jax.ShapeDtypeStruct((N, D_IN), jnp.float32),
    )(rd, mf, bs, feat, xg, vecs, w2, w3p, ebmat, wq, wout)


def kernel(features_0, neighbor_indices, neighbor_mask, rel_dist, basis_0_0,
           Wq, W1, b1, g1, be1, W2, b2, g2, be2, W3, b3, Wout):
    feat = features_0.reshape(N, D_IN)
    idx3 = neighbor_indices.reshape(EDGES).astype(jnp.int32).reshape(NW, CH, CW)
    rd = rel_dist.reshape(N, K)
    mf = neighbor_mask.reshape(N, K).astype(jnp.float32)
    bs = basis_0_0.reshape(N, K)

    a = W1.reshape(MID)
    am = a - jnp.mean(a)
    cm = b1 - jnp.mean(b1)
    vecs = jnp.concatenate(
        [jnp.stack([am, cm, g1, be1, b2, g2, be2], axis=0),
         jnp.zeros((1, MID), jnp.float32)], axis=0)
    w3p = W3.reshape(MID, KV_DIM, D_IN).transpose(0, 2, 1).reshape(
        MID, KV_DIM * D_IN)
    b3m = b3.reshape(KV_DIM, D_IN).T
    emat = jnp.kron(jnp.eye(D_IN, dtype=jnp.float32),
                    jnp.ones((1, KV_DIM), jnp.float32))
    ebmat = jnp.concatenate([emat, b3m], axis=1)

    gathered = _sc_gather(feat, idx3)
    out = _tc_fused(rd, mf, bs, feat, gathered, vecs, W2, w3p, ebmat, Wq,
                    Wout)
    return out.reshape(1, N, HIDDEN, 1)


# speedup vs baseline: 3.0585x; 1.2302x over previous
import functools

import jax
import jax.numpy as jnp
from jax import lax
from jax.experimental import pallas as pl
from jax.experimental.pallas import tpu as pltpu
from jax.experimental.pallas import tpu_sc as plsc

N = 10000
K = 16
D_IN = 16
HEADS = 2
DIM_HEAD = 8
HIDDEN = HEADS * DIM_HEAD
KV_DIM = HIDDEN * 2
MID = 128
EDGES = N * K
EPS = 1e-5

NUM_CORES = 2
NUM_SUBCORES = 16
NW = NUM_CORES * NUM_SUBCORES
PER_W = EDGES // NW
CW = 125
CH = PER_W // CW

T = 400
GRID = N // T
ET = T * K


def _sc_gather(table, idx3):
    mesh = plsc.VectorSubcoreMesh(core_axis_name="c", subcore_axis_name="s")

    @functools.partial(
        pl.kernel,
        mesh=mesh,
        out_type=jax.ShapeDtypeStruct((EDGES, D_IN), jnp.float32),
        scratch_types=[
            pltpu.VMEM((CH, CW), jnp.int32),
            pltpu.VMEM((PER_W, D_IN), jnp.float32),
            pltpu.SemaphoreType.DMA,
        ],
        compiler_params=pltpu.CompilerParams(use_tc_tiling_on_sc=False),
    )
    def gather_kernel(table_hbm, idx_hbm, out_hbm, idx_v, rows_v, sem):
        wid = lax.axis_index("s") * NUM_CORES + lax.axis_index("c")
        pltpu.sync_copy(idx_hbm.at[wid], idx_v)

        def body(ci, carry):
            pltpu.async_copy(
                table_hbm.at[idx_v.at[ci]],
                rows_v.at[pl.ds(ci * CW, CW)],
                sem,
            ).wait()
            return carry

        lax.fori_loop(0, CH, body, 0)
        pltpu.sync_copy(rows_v, out_hbm.at[pl.ds(wid * PER_W, PER_W)])

    return gather_kernel(table, idx3)


def _tc_body(rd_ref, mf_ref, bs_ref, feat_ref, xg_ref, vecs_ref, w2_ref,
             w3_ref, eb_ref, wq_ref, wout_ref, out_ref):
    f32 = jnp.float32
    rd = rd_ref[...]
    am = vecs_ref[0:1, :].reshape(1, 1, MID)
    cm = vecs_ref[1:2, :].reshape(1, 1, MID)
    g1 = vecs_ref[2:3, :].reshape(1, 1, MID)
    be1 = vecs_ref[3:4, :].reshape(1, 1, MID)
    pre = rd[:, :, None] * am + cm
    var1 = jnp.mean(pre * pre, axis=-1, keepdims=True)
    h1 = jnp.maximum(pre * lax.rsqrt(var1 + EPS) * g1 + be1, 0.0)
    h1f = h1.reshape(ET, MID)
    h2p = jnp.dot(h1f, w2_ref[...], preferred_element_type=f32)
    h2p = h2p + vecs_ref[4:5, :]
    m2 = jnp.mean(h2p, axis=-1, keepdims=True)
    d2 = h2p - m2
    var2 = jnp.mean(d2 * d2, axis=-1, keepdims=True)
    h2 = jnp.maximum(d2 * lax.rsqrt(var2 + EPS) * vecs_ref[5:6, :]
                     + vecs_ref[6:7, :], 0.0)
    y = jnp.dot(h2, w3_ref[...], preferred_element_type=f32)
    xg = xg_ref[...]
    eb = eb_ref[...]
    xh = lax.convert_element_type(
        lax.convert_element_type(xg, jnp.bfloat16), f32)
    xl = xg - xh
    mb = (jnp.dot(xh, eb, preferred_element_type=f32)
          + jnp.dot(xl, eb, preferred_element_type=f32))
    z = y * mb[:, :KV_DIM * D_IN]
    k128 = ((z[:, 0:MID] + z[:, MID:2 * MID])
            + (z[:, 2 * MID:3 * MID] + z[:, 3 * MID:4 * MID]))
    kv = ((k128[:, 0:KV_DIM] + k128[:, KV_DIM:2 * KV_DIM])
          + (k128[:, 2 * KV_DIM:3 * KV_DIM] + k128[:, 3 * KV_DIM:4 * KV_DIM]))
    kv = kv + mb[:, KV_DIM * D_IN:]
    kv3 = kv.reshape(T, K, KV_DIM) * bs_ref[...][:, :, None]
    q = jnp.dot(feat_ref[...], wq_ref[...], preferred_element_type=f32)
    mf = mf_ref[...]
    scale = DIM_HEAD ** -0.5
    neg = -jnp.finfo(f32).max
    outs = []
    for h in range(HEADS):
        qh = q[:, h * DIM_HEAD:(h + 1) * DIM_HEAD]
        kh = kv3[:, :, h * 2 * DIM_HEAD:h * 2 * DIM_HEAD + DIM_HEAD]
        vh = kv3[:, :, h * 2 * DIM_HEAD + DIM_HEAD:(h + 1) * 2 * DIM_HEAD]
        sim = jnp.sum(qh[:, None, :] * kh, axis=-1) * scale
        sim = jnp.where(mf != 0.0, sim, neg)
        mx = jnp.max(sim, axis=-1, keepdims=True)
        ex = jnp.exp(sim - mx)
        attn = ex / jnp.sum(ex, axis=-1, keepdims=True)
        outs.append(jnp.sum(attn[:, :, None] * vh, axis=1))
    out16 = jnp.concatenate(outs, axis=-1)
    out_ref[...] = jnp.dot(out16, wout_ref[...], preferred_element_type=f32)


def _tc_fused(rd, mf, bs, feat, xg, vecs, w2, w3p, ebmat, wq, wout):
    node_spec = pl.BlockSpec((T, K), lambda i: (i, 0))
    full = lambda shape: pl.BlockSpec(shape, lambda i: tuple(0 for _ in shape))
    return pl.pallas_call(
        _tc_body,
        grid=(GRID,),
        in_specs=[
            node_spec,
            node_spec,
            node_spec,
            pl.BlockSpec((T, D_IN), lambda i: (i, 0)),
            pl.BlockSpec((ET, D_IN), lambda i: (i, 0)),
            full((8, MID)),
            full((MID, MID)),
            full((MID, KV_DIM * D_IN)),
            full((D_IN, KV_DIM * D_IN + KV_DIM)),
            full((D_IN, HIDDEN)),
            full((HIDDEN, D_IN)),
        ],
        out_specs=pl.BlockSpec((T, D_IN), lambda i: (i, 0)),
        out_shape=jax.ShapeDtypeStruct((N, D_IN), jnp.float32),
    )(rd, mf, bs, feat, xg, vecs, w2, w3p, ebmat, wq, wout)


def kernel(features_0, neighbor_indices, neighbor_mask, rel_dist, basis_0_0,
           Wq, W1, b1, g1, be1, W2, b2, g2, be2, W3, b3, Wout):
    feat = features_0.reshape(N, D_IN)
    idx3 = neighbor_indices.reshape(EDGES).astype(jnp.int32).reshape(NW, CH, CW)
    rd = rel_dist.reshape(N, K)
    mf = neighbor_mask.reshape(N, K).astype(jnp.float32)
    bs = basis_0_0.reshape(N, K)

    a = W1.reshape(MID)
    am = a - jnp.mean(a)
    cm = b1 - jnp.mean(b1)
    vecs = jnp.concatenate(
        [jnp.stack([am, cm, g1, be1, b2, g2, be2], axis=0),
         jnp.zeros((1, MID), jnp.float32)], axis=0)
    w3p = W3.reshape(MID, KV_DIM, D_IN).transpose(0, 2, 1).reshape(
        MID, KV_DIM * D_IN)
    b3m = b3.reshape(KV_DIM, D_IN).T
    emat = jnp.kron(jnp.eye(D_IN, dtype=jnp.float32),
                    jnp.ones((1, KV_DIM), jnp.float32))
    ebmat = jnp.concatenate([emat, b3m], axis=1)

    gathered = _sc_gather(feat, idx3)
    out = _tc_fused(rd, mf, bs, feat, gathered, vecs, W2, w3p, ebmat, Wq,
                    Wout)
    return out.reshape(1, N, HIDDEN, 1)


# speedup vs baseline: 3.0628x; 1.0014x over previous
import functools

import jax
import jax.numpy as jnp
from jax import lax
from jax.experimental import pallas as pl
from jax.experimental.pallas import tpu as pltpu
from jax.experimental.pallas import tpu_sc as plsc

N = 10000
K = 16
D_IN = 16
HEADS = 2
DIM_HEAD = 8
HIDDEN = HEADS * DIM_HEAD
KV_DIM = HIDDEN * 2
MID = 128
EDGES = N * K
EPS = 1e-5

NUM_CORES = 2
NUM_SUBCORES = 16
NW = NUM_CORES * NUM_SUBCORES
PER_W = EDGES // NW
CW = 125
CH = PER_W // CW

T = 400
GRID = N // T
ET = T * K


def _sc_gather(table, idx3):
    mesh = plsc.VectorSubcoreMesh(core_axis_name="c", subcore_axis_name="s")

    @functools.partial(
        pl.kernel,
        mesh=mesh,
        out_type=jax.ShapeDtypeStruct((EDGES, D_IN), jnp.float32),
        scratch_types=[
            pltpu.VMEM((CH, CW), jnp.int32),
            pltpu.VMEM((PER_W, D_IN), jnp.float32),
            pltpu.SemaphoreType.DMA,
        ],
        compiler_params=pltpu.CompilerParams(use_tc_tiling_on_sc=False),
    )
    def gather_kernel(table_hbm, idx_hbm, out_hbm, idx_v, rows_v, sem):
        wid = lax.axis_index("s") * NUM_CORES + lax.axis_index("c")
        pltpu.sync_copy(idx_hbm.at[wid], idx_v)

        def body(ci, carry):
            pltpu.async_copy(
                table_hbm.at[idx_v.at[ci]],
                rows_v.at[pl.ds(ci * CW, CW)],
                sem,
            ).wait()
            return carry

        lax.fori_loop(0, CH, body, 0)
        pltpu.sync_copy(rows_v, out_hbm.at[pl.ds(wid * PER_W, PER_W)])

    return gather_kernel(table, idx3)


def _tc_body(rd_ref, mf_ref, bs_ref, feat_ref, xg_ref, vecs_ref, w2_ref,
             w3_ref, eb_ref, wq_ref, wout_ref, out_ref):
    f32 = jnp.float32
    rd = rd_ref[...]
    am = vecs_ref[0:1, :].reshape(1, 1, MID)
    cm = vecs_ref[1:2, :].reshape(1, 1, MID)
    g1 = vecs_ref[2:3, :].reshape(1, 1, MID)
    be1 = vecs_ref[3:4, :].reshape(1, 1, MID)
    pre = rd[:, :, None] * am + cm
    var1 = jnp.mean(pre * pre, axis=-1, keepdims=True)
    h1 = jnp.maximum(pre * lax.rsqrt(var1 + EPS) * g1 + be1, 0.0)
    h1f = h1.reshape(ET, MID)
    h2p = jnp.dot(h1f, w2_ref[...], preferred_element_type=f32)
    h2p = h2p + vecs_ref[4:5, :]
    m2 = jnp.mean(h2p, axis=-1, keepdims=True)
    d2 = h2p - m2
    var2 = jnp.mean(d2 * d2, axis=-1, keepdims=True)
    h2 = jnp.maximum(d2 * lax.rsqrt(var2 + EPS) * vecs_ref[5:6, :]
                     + vecs_ref[6:7, :], 0.0)
    y = jnp.dot(h2, w3_ref[...], preferred_element_type=f32)
    xg = xg_ref[...]
    eb = eb_ref[...]
    xh = lax.convert_element_type(xg, jnp.bfloat16)
    xl = lax.convert_element_type(
        xg - lax.convert_element_type(xh, f32), jnp.bfloat16)
    mb = (jnp.dot(xh, eb, preferred_element_type=f32)
          + jnp.dot(xl, eb, preferred_element_type=f32))
    z = y * mb[:, :KV_DIM * D_IN]
    k128 = ((z[:, 0:MID] + z[:, MID:2 * MID])
            + (z[:, 2 * MID:3 * MID] + z[:, 3 * MID:4 * MID]))
    kv = ((k128[:, 0:KV_DIM] + k128[:, KV_DIM:2 * KV_DIM])
          + (k128[:, 2 * KV_DIM:3 * KV_DIM] + k128[:, 3 * KV_DIM:4 * KV_DIM]))
    kv = kv + mb[:, KV_DIM * D_IN:]
    kv3 = kv.reshape(T, K, KV_DIM) * bs_ref[...][:, :, None]
    q = jnp.dot(feat_ref[...], wq_ref[...], preferred_element_type=f32)
    mf = mf_ref[...]
    scale = DIM_HEAD ** -0.5
    neg = -jnp.finfo(f32).max
    outs = []
    for h in range(HEADS):
        qh = q[:, h * DIM_HEAD:(h + 1) * DIM_HEAD]
        kh = kv3[:, :, h * 2 * DIM_HEAD:h * 2 * DIM_HEAD + DIM_HEAD]
        vh = kv3[:, :, h * 2 * DIM_HEAD + DIM_HEAD:(h + 1) * 2 * DIM_HEAD]
        sim = jnp.sum(qh[:, None, :] * kh, axis=-1) * scale
        sim = jnp.where(mf != 0.0, sim, neg)
        mx = jnp.max(sim, axis=-1, keepdims=True)
        ex = jnp.exp(sim - mx)
        attn = ex / jnp.sum(ex, axis=-1, keepdims=True)
        outs.append(jnp.sum(attn[:, :, None] * vh, axis=1))
    out16 = jnp.concatenate(outs, axis=-1)
    out_ref[...] = jnp.dot(out16, wout_ref[...], preferred_element_type=f32)


def _tc_fused(rd, mf, bs, feat, xg, vecs, w2, w3p, ebmat, wq, wout):
    node_spec = pl.BlockSpec((T, K), lambda i: (i, 0))
    full = lambda shape: pl.BlockSpec(shape, lambda i: tuple(0 for _ in shape))
    return pl.pallas_call(
        _tc_body,
        grid=(GRID,),
        in_specs=[
            node_spec,
            node_spec,
            node_spec,
            pl.BlockSpec((T, D_IN), lambda i: (i, 0)),
            pl.BlockSpec((ET, D_IN), lambda i: (i, 0)),
            full((8, MID)),
            full((MID, MID)),
            full((MID, KV_DIM * D_IN)),
            full((D_IN, KV_DIM * D_IN + KV_DIM)),
            full((D_IN, HIDDEN)),
            full((HIDDEN, D_IN)),
        ],
        out_specs=pl.BlockSpec((T, D_IN), lambda i: (i, 0)),
        out_shape=jax.ShapeDtypeStruct((N, D_IN), jnp.float32),
    )(rd, mf, bs, feat, xg, vecs, w2, w3p, ebmat, wq, wout)


def kernel(features_0, neighbor_indices, neighbor_mask, rel_dist, basis_0_0,
           Wq, W1, b1, g1, be1, W2, b2, g2, be2, W3, b3, Wout):
    feat = features_0.reshape(N, D_IN)
    idx3 = neighbor_indices.reshape(EDGES).astype(jnp.int32).reshape(NW, CH, CW)
    rd = rel_dist.reshape(N, K)
    mf = neighbor_mask.reshape(N, K).astype(jnp.float32)
    bs = basis_0_0.reshape(N, K)

    a = W1.reshape(MID)
    am = a - jnp.mean(a)
    cm = b1 - jnp.mean(b1)
    vecs = jnp.concatenate(
        [jnp.stack([am, cm, g1, be1, b2, g2, be2], axis=0),
         jnp.zeros((1, MID), jnp.float32)], axis=0)
    w3p = W3.reshape(MID, KV_DIM, D_IN).transpose(0, 2, 1).reshape(
        MID, KV_DIM * D_IN)
    b3m = b3.reshape(KV_DIM, D_IN).T
    emat = jnp.kron(jnp.eye(D_IN, dtype=jnp.float32),
                    jnp.ones((1, KV_DIM), jnp.float32))
    ebmat = jnp.concatenate([emat, b3m], axis=1).astype(jnp.bfloat16)

    gathered = _sc_gather(feat, idx3)
    out = _tc_fused(rd, mf, bs, feat, gathered, vecs, W2, w3p, ebmat, Wq,
                    Wout)
    return out.reshape(1, N, HIDDEN, 1)


# speedup vs baseline: 3.2288x; 1.0542x over previous
import functools

import numpy as np

import jax
import jax.numpy as jnp
from jax import lax
from jax.experimental import pallas as pl
from jax.experimental.pallas import tpu as pltpu
from jax.experimental.pallas import tpu_sc as plsc

N = 10000
K = 16
D_IN = 16
HEADS = 2
DIM_HEAD = 8
HIDDEN = HEADS * DIM_HEAD
KV_DIM = HIDDEN * 2
MID = 128
EDGES = N * K
EPS = 1e-5

NUM_CORES = 2
NUM_SUBCORES = 16
NW = NUM_CORES * NUM_SUBCORES
PER_W = EDGES // NW
CW = 125
CH = PER_W // CW

T = 400
GRID = N // T
ET = T * K


def _sc_gather(table, idx3):
    mesh = plsc.VectorSubcoreMesh(core_axis_name="c", subcore_axis_name="s")

    @functools.partial(
        pl.kernel,
        mesh=mesh,
        out_type=jax.ShapeDtypeStruct((EDGES, D_IN), jnp.float32),
        scratch_types=[
            pltpu.VMEM((CH, CW), jnp.int32),
            pltpu.VMEM((PER_W, D_IN), jnp.float32),
            pltpu.SemaphoreType.DMA,
        ],
        compiler_params=pltpu.CompilerParams(use_tc_tiling_on_sc=False),
    )
    def gather_kernel(table_hbm, idx_hbm, out_hbm, idx_v, rows_v, sem):
        wid = lax.axis_index("s") * NUM_CORES + lax.axis_index("c")
        pltpu.sync_copy(idx_hbm.at[wid], idx_v)

        def body(ci, carry):
            pltpu.async_copy(
                table_hbm.at[idx_v.at[ci]],
                rows_v.at[pl.ds(ci * CW, CW)],
                sem,
            ).wait()
            return carry

        lax.fori_loop(0, CH, body, 0)
        pltpu.sync_copy(rows_v, out_hbm.at[pl.ds(wid * PER_W, PER_W)])

    return gather_kernel(table, idx3)


def _split2(x, f32):
    xh = lax.convert_element_type(x, jnp.bfloat16)
    xl = lax.convert_element_type(
        x - lax.convert_element_type(xh, f32), jnp.bfloat16)
    return xh, xl


def _dot2(x, w, f32):
    xh, xl = _split2(x, f32)
    return (jnp.dot(xh, w, preferred_element_type=f32)
            + jnp.dot(xl, w, preferred_element_type=f32))


def _tc_body(scal_ref, rd_ref, mf_ref, bs_ref, feat_ref, xg_ref, vecs_ref,
             w2_ref, w3_ref, eb_ref, b3m_ref, wq_ref, wout_ref, out_ref):
    f32 = jnp.float32
    rd = rd_ref[...]
    pg = vecs_ref[0:1, :].reshape(1, 1, MID)
    qg = vecs_ref[1:2, :].reshape(1, 1, MID)
    be1 = vecs_ref[2:3, :].reshape(1, 1, MID)
    inv = lax.rsqrt(scal_ref[0] * rd * rd + scal_ref[1] * rd
                    + scal_ref[2] + EPS)
    u = rd * inv
    h1 = jnp.maximum(u[:, :, None] * pg + inv[:, :, None] * qg + be1, 0.0)
    h1f = h1.reshape(ET, MID)
    h2p = jnp.dot(h1f, w2_ref[...], preferred_element_type=f32)
    h2p = h2p + vecs_ref[3:4, :]
    m2 = jnp.mean(h2p, axis=-1, keepdims=True)
    d2 = h2p - m2
    var2 = jnp.mean(d2 * d2, axis=-1, keepdims=True)
    h2 = jnp.maximum(d2 * lax.rsqrt(var2 + EPS) * vecs_ref[4:5, :]
                     + vecs_ref[5:6, :], 0.0)
    y = jnp.dot(h2, w3_ref[...], preferred_element_type=f32)
    xg = xg_ref[...]
    eb = eb_ref[...]
    xh, xl = _split2(xg, f32)
    mb = (jnp.dot(xh, eb, preferred_element_type=f32)
          + jnp.dot(xl, eb, preferred_element_type=f32))
    z = y * mb
    k128 = ((z[:, 0:MID] + z[:, MID:2 * MID])
            + (z[:, 2 * MID:3 * MID] + z[:, 3 * MID:4 * MID]))
    kv = ((k128[:, 0:KV_DIM] + k128[:, KV_DIM:2 * KV_DIM])
          + (k128[:, 2 * KV_DIM:3 * KV_DIM] + k128[:, 3 * KV_DIM:4 * KV_DIM]))
    kv = kv + jnp.dot(xg, b3m_ref[...], preferred_element_type=f32)
    kv3 = kv.reshape(T, K, KV_DIM) * bs_ref[...][:, :, None]
    q = jnp.dot(feat_ref[...], wq_ref[...], preferred_element_type=f32)
    mf = mf_ref[...]
    scale = DIM_HEAD ** -0.5
    neg = -jnp.finfo(f32).max
    outs = []
    for h in range(HEADS):
        qh = q[:, h * DIM_HEAD:(h + 1) * DIM_HEAD]
        kh = kv3[:, :, h * 2 * DIM_HEAD:h * 2 * DIM_HEAD + DIM_HEAD]
        vh = kv3[:, :, h * 2 * DIM_HEAD + DIM_HEAD:(h + 1) * 2 * DIM_HEAD]
        sim = jnp.sum(qh[:, None, :] * kh, axis=-1) * scale
        sim = jnp.where(mf != 0.0, sim, neg)
        mx = jnp.max(sim, axis=-1, keepdims=True)
        ex = jnp.exp(sim - mx)
        attn = ex / jnp.sum(ex, axis=-1, keepdims=True)
        outs.append(jnp.sum(attn[:, :, None] * vh, axis=1))
    out16 = jnp.concatenate(outs, axis=-1)
    out_ref[...] = jnp.dot(out16, wout_ref[...], preferred_element_type=f32)


def _tc_fused(scal, rd, mf, bs, feat, xg, vecs, w2, w3p, ebmat, b3m, wq,
              wout):
    node_spec = pl.BlockSpec((T, K), lambda i: (i, 0))
    full = lambda shape: pl.BlockSpec(shape, lambda i: tuple(0 for _ in shape))
    return pl.pallas_call(
        _tc_body,
        grid=(GRID,),
        in_specs=[
            pl.BlockSpec(memory_space=pltpu.SMEM),
            node_spec,
            node_spec,
            node_spec,
            pl.BlockSpec((T, D_IN), lambda i: (i, 0)),
            pl.BlockSpec((ET, D_IN), lambda i: (i, 0)),
            full((8, MID)),
            full((MID, MID)),
            full((MID, KV_DIM * D_IN)),
            full((D_IN, KV_DIM * D_IN)),
            full((D_IN, KV_DIM)),
            full((D_IN, HIDDEN)),
            full((HIDDEN, D_IN)),
        ],
        out_specs=pl.BlockSpec((T, D_IN), lambda i: (i, 0)),
        out_shape=jax.ShapeDtypeStruct((N, D_IN), jnp.float32),
    )(scal, rd, mf, bs, feat, xg, vecs, w2, w3p, ebmat, b3m, wq, wout)


def kernel(features_0, neighbor_indices, neighbor_mask, rel_dist, basis_0_0,
           Wq, W1, b1, g1, be1, W2, b2, g2, be2, W3, b3, Wout):
    feat = features_0.reshape(N, D_IN)
    idx3 = neighbor_indices.reshape(EDGES).astype(jnp.int32).reshape(NW, CH, CW)
    rd = rel_dist.reshape(N, K)
    mf = neighbor_mask.reshape(N, K).astype(jnp.float32)
    bs = basis_0_0.reshape(N, K)

    a = W1.reshape(MID)
    am = a - jnp.mean(a)
    cm = b1 - jnp.mean(b1)
    scal = jnp.stack([jnp.mean(am * am), 2.0 * jnp.mean(am * cm),
                      jnp.mean(cm * cm)])
    vecs = jnp.concatenate(
        [jnp.stack([am * g1, cm * g1, be1, b2, g2, be2], axis=0),
         jnp.zeros((2, MID), jnp.float32)], axis=0)
    w3p = W3.reshape(MID, KV_DIM, D_IN).transpose(0, 2, 1).reshape(
        MID, KV_DIM * D_IN)
    b3m = b3.reshape(KV_DIM, D_IN).T
    ebmat = jnp.kron(jnp.eye(D_IN, dtype=jnp.float32),
                     jnp.ones((1, KV_DIM), jnp.float32)
                     ).astype(jnp.bfloat16)

    gathered = _sc_gather(feat, idx3)
    out = _tc_fused(scal, rd, mf, bs, feat, gathered, vecs, W2, w3p, ebmat,
                    b3m, Wq, Wout)
    return out.reshape(1, N, HIDDEN, 1)


# speedup vs baseline: 3.3359x; 1.0332x over previous
import functools

import numpy as np

import jax
import jax.numpy as jnp
from jax import lax
from jax.experimental import pallas as pl
from jax.experimental.pallas import tpu as pltpu
from jax.experimental.pallas import tpu_sc as plsc

N = 10000
K = 16
D_IN = 16
HEADS = 2
DIM_HEAD = 8
HIDDEN = HEADS * DIM_HEAD
KV_DIM = HIDDEN * 2
MID = 128
EDGES = N * K
EPS = 1e-5

NUM_CORES = 2
NUM_SUBCORES = 16
NW = NUM_CORES * NUM_SUBCORES
PER_W = EDGES // NW
CW = 125
CH = PER_W // CW

T = 400
GRID = N // T
ET = T * K


def _sc_gather(table, idx3):
    mesh = plsc.VectorSubcoreMesh(core_axis_name="c", subcore_axis_name="s")

    @functools.partial(
        pl.kernel,
        mesh=mesh,
        out_type=jax.ShapeDtypeStruct((EDGES, D_IN), jnp.float32),
        scratch_types=[
            pltpu.VMEM((CH, CW), jnp.int32),
            pltpu.VMEM((PER_W, D_IN), jnp.float32),
            pltpu.SemaphoreType.DMA,
        ],
        compiler_params=pltpu.CompilerParams(use_tc_tiling_on_sc=False),
    )
    def gather_kernel(table_hbm, idx_hbm, out_hbm, idx_v, rows_v, sem):
        wid = lax.axis_index("s") * NUM_CORES + lax.axis_index("c")
        pltpu.sync_copy(idx_hbm.at[wid], idx_v)

        def fire(ci, carry):
            pltpu.async_copy(
                table_hbm.at[idx_v.at[ci]],
                rows_v.at[pl.ds(ci * CW, CW)],
                sem,
            )
            return carry

        def drain(ci, carry):
            pltpu.make_async_copy(
                table_hbm.at[idx_v.at[ci]],
                rows_v.at[pl.ds(ci * CW, CW)],
                sem,
            ).wait()
            return carry

        lax.fori_loop(0, CH, fire, 0)
        lax.fori_loop(0, CH, drain, 0)
        pltpu.sync_copy(rows_v, out_hbm.at[pl.ds(wid * PER_W, PER_W)])

    return gather_kernel(table, idx3)


def _split2(x, f32):
    xh = lax.convert_element_type(x, jnp.bfloat16)
    xl = lax.convert_element_type(
        x - lax.convert_element_type(xh, f32), jnp.bfloat16)
    return xh, xl


def _dot2(x, w, f32):
    xh, xl = _split2(x, f32)
    return (jnp.dot(xh, w, preferred_element_type=f32)
            + jnp.dot(xl, w, preferred_element_type=f32))


def _tc_body(scal_ref, rd_ref, mf_ref, bs_ref, feat_ref, xg_ref, vecs_ref,
             w2_ref, w3_ref, eb_ref, b3m_ref, wq_ref, wout_ref, out_ref):
    f32 = jnp.float32
    rd = rd_ref[...]
    pg = vecs_ref[0:1, :].reshape(1, 1, MID)
    qg = vecs_ref[1:2, :].reshape(1, 1, MID)
    be1 = vecs_ref[2:3, :].reshape(1, 1, MID)
    inv = lax.rsqrt(scal_ref[0] * rd * rd + scal_ref[1] * rd
                    + scal_ref[2] + EPS)
    u = rd * inv
    h1 = jnp.maximum(u[:, :, None] * pg + inv[:, :, None] * qg + be1, 0.0)
    h1f = h1.reshape(ET, MID)
    h2p = jnp.dot(h1f, w2_ref[...], preferred_element_type=f32)
    h2p = h2p + vecs_ref[3:4, :]
    m2 = jnp.mean(h2p, axis=-1, keepdims=True)
    d2 = h2p - m2
    var2 = jnp.mean(d2 * d2, axis=-1, keepdims=True)
    h2 = jnp.maximum(d2 * lax.rsqrt(var2 + EPS) * vecs_ref[4:5, :]
                     + vecs_ref[5:6, :], 0.0)
    y = jnp.dot(h2, w3_ref[...], preferred_element_type=f32)
    xg = xg_ref[...]
    eb = eb_ref[...]
    xh, xl = _split2(xg, f32)
    mb = (jnp.dot(xh, eb, preferred_element_type=f32)
          + jnp.dot(xl, eb, preferred_element_type=f32))
    z = y * mb
    k128 = ((z[:, 0:MID] + z[:, MID:2 * MID])
            + (z[:, 2 * MID:3 * MID] + z[:, 3 * MID:4 * MID]))
    kv = ((k128[:, 0:KV_DIM] + k128[:, KV_DIM:2 * KV_DIM])
          + (k128[:, 2 * KV_DIM:3 * KV_DIM] + k128[:, 3 * KV_DIM:4 * KV_DIM]))
    kv = kv + jnp.dot(xg, b3m_ref[...], preferred_element_type=f32)
    kv3 = kv.reshape(T, K, KV_DIM) * bs_ref[...][:, :, None]
    q = jnp.dot(feat_ref[...], wq_ref[...], preferred_element_type=f32)
    mf = mf_ref[...]
    scale = DIM_HEAD ** -0.5
    neg = -jnp.finfo(f32).max
    outs = []
    for h in range(HEADS):
        qh = q[:, h * DIM_HEAD:(h + 1) * DIM_HEAD]
        kh = kv3[:, :, h * 2 * DIM_HEAD:h * 2 * DIM_HEAD + DIM_HEAD]
        vh = kv3[:, :, h * 2 * DIM_HEAD + DIM_HEAD:(h + 1) * 2 * DIM_HEAD]
        sim = jnp.sum(qh[:, None, :] * kh, axis=-1) * scale
        sim = jnp.where(mf != 0.0, sim, neg)
        mx = jnp.max(sim, axis=-1, keepdims=True)
        ex = jnp.exp(sim - mx)
        attn = ex / jnp.sum(ex, axis=-1, keepdims=True)
        outs.append(jnp.sum(attn[:, :, None] * vh, axis=1))
    out16 = jnp.concatenate(outs, axis=-1)
    out_ref[...] = jnp.dot(out16, wout_ref[...], preferred_element_type=f32)


def _tc_fused(scal, rd, mf, bs, feat, xg, vecs, w2, w3p, ebmat, b3m, wq,
              wout):
    node_spec = pl.BlockSpec((T, K), lambda i: (i, 0))
    full = lambda shape: pl.BlockSpec(shape, lambda i: tuple(0 for _ in shape))
    return pl.pallas_call(
        _tc_body,
        grid=(GRID,),
        in_specs=[
            pl.BlockSpec(memory_space=pltpu.SMEM),
            node_spec,
            node_spec,
            node_spec,
            pl.BlockSpec((T, D_IN), lambda i: (i, 0)),
            pl.BlockSpec((ET, D_IN), lambda i: (i, 0)),
            full((8, MID)),
            full((MID, MID)),
            full((MID, KV_DIM * D_IN)),
            full((D_IN, KV_DIM * D_IN)),
            full((D_IN, KV_DIM)),
            full((D_IN, HIDDEN)),
            full((HIDDEN, D_IN)),
        ],
        out_specs=pl.BlockSpec((T, D_IN), lambda i: (i, 0)),
        out_shape=jax.ShapeDtypeStruct((N, D_IN), jnp.float32),
    )(scal, rd, mf, bs, feat, xg, vecs, w2, w3p, ebmat, b3m, wq, wout)


def kernel(features_0, neighbor_indices, neighbor_mask, rel_dist, basis_0_0,
           Wq, W1, b1, g1, be1, W2, b2, g2, be2, W3, b3, Wout):
    feat = features_0.reshape(N, D_IN)
    idx3 = neighbor_indices.reshape(EDGES).astype(jnp.int32).reshape(NW, CH, CW)
    rd = rel_dist.reshape(N, K)
    mf = neighbor_mask.reshape(N, K).astype(jnp.float32)
    bs = basis_0_0.reshape(N, K)

    a = W1.reshape(MID)
    am = a - jnp.mean(a)
    cm = b1 - jnp.mean(b1)
    scal = jnp.stack([jnp.mean(am * am), 2.0 * jnp.mean(am * cm),
                      jnp.mean(cm * cm)])
    vecs = jnp.concatenate(
        [jnp.stack([am * g1, cm * g1, be1, b2, g2, be2], axis=0),
         jnp.zeros((2, MID), jnp.float32)], axis=0)
    w3p = W3.reshape(MID, KV_DIM, D_IN).transpose(0, 2, 1).reshape(
        MID, KV_DIM * D_IN)
    b3m = b3.reshape(KV_DIM, D_IN).T
    ebmat = jnp.kron(jnp.eye(D_IN, dtype=jnp.float32),
                     jnp.ones((1, KV_DIM), jnp.float32)
                     ).astype(jnp.bfloat16)

    gathered = _sc_gather(feat, idx3)
    out = _tc_fused(scal, rd, mf, bs, feat, gathered, vecs, W2, w3p, ebmat,
                    b3m, Wq, Wout)
    return out.reshape(1, N, HIDDEN, 1)
